# baseline (speedup 1.0000x reference)
import functools

import jax
import jax.numpy as jnp
import numpy as np
from jax.experimental import pallas as pl

B, N, C = 4, 2048, 192
N0, Ns = 4096, 2048
H, W = 64, 64
NUM_HEADS = 8
HD = C // NUM_HEADS
HWW = 7
NH = 10
PAD_OFF = 3
G = H * W
NW = NH * NH
WPAD = 128
CE = 256


SLOTS = 64
GW = 7168


def _win_of_grid():
    t = np.arange(G)
    y, x = t // W, t % W
    w = ((y + PAD_OFF) // HWW) * NH + (x + PAD_OFF) // HWW
    return w.astype(np.int32).reshape(1, 1, G)


def _cell_of_slot():
    m = np.full((GW,), G, np.int32)
    for w in range(NW):
        wy, wx = w // NH, w % NH
        for j in range(HWW * HWW):
            jy, jx = j // HWW, j % HWW
            y, x = wy * HWW + jy - PAD_OFF, wx * HWW + jx - PAD_OFF
            if 0 <= y < H and 0 <= x < W:
                m[w * SLOTS + j] = y * W + x
    return m


def _win_of_slot():
    w = np.minimum(np.arange(GW, dtype=np.int32) // SLOTS, NW - 1)
    w[_cell_of_slot() == G] = -1
    return w.reshape(1, 1, GW)


def _slot_of_cell():
    cm = _cell_of_slot()
    inv = np.zeros((G,), np.int32)
    inv[cm[cm < G]] = np.nonzero(cm < G)[0].astype(np.int32)
    return inv


_RBLK = 256


def _route_body(idxw_ref, agg_ref, aw_ref, out_ref):
    nb = pl.program_id(1)
    agg = agg_ref[0]
    aw = aw_ref[0]
    n_iota = jax.lax.broadcasted_iota(jnp.int32, (_RBLK, N0), 0) + nb * _RBLK
    oh_n = (agg == n_iota).astype(jnp.float32)
    iw = idxw_ref[0]
    w_iota = jax.lax.broadcasted_iota(jnp.int32, (N0, WPAD), 1)
    wv = jnp.where(iw == w_iota, aw, 0.0)
    votes = jax.lax.dot_general(oh_n, wv, (((1,), (0,)), ((), ())),
                                precision=jax.lax.Precision.HIGHEST,
                                preferred_element_type=jnp.float32)
    m = jnp.max(votes, axis=1, keepdims=True)
    cand = jnp.where(votes == m,
                     jax.lax.broadcasted_iota(jnp.int32, (_RBLK, WPAD), 1),
                     jnp.int32(2 ** 30))
    out_ref[0] = jnp.min(cand, axis=1, keepdims=True)


def _route(idx_tmp, agg, aw):
    grid = (B, N // _RBLK)
    return pl.pallas_call(
        _route_body,
        grid=grid,
        in_specs=[
            pl.BlockSpec((1, N0, 1), lambda b, n: (b, 0, 0)),
            pl.BlockSpec((1, 1, N0), lambda b, n: (b, 0, 0)),
            pl.BlockSpec((1, N0, 1), lambda b, n: (b, 0, 0)),
        ],
        out_specs=pl.BlockSpec((1, _RBLK, 1), lambda b, n: (b, n, 0)),
        out_shape=jax.ShapeDtypeStruct((B, N, 1), jnp.int32),
    )(idx_tmp, agg, aw)


_TCHUNK = 512
_NCHUNK = N0 // _TCHUNK


def _t2m_body(sidx_ref, ihw_ref, src_ref, out_ref):
    c = pl.program_id(1)

    @pl.when(c == 0)
    def _init():
        out_ref[0] = jnp.zeros((GW, CE), jnp.float32)

    sidx = sidx_ref[0]
    ihw = ihw_ref[0]
    src = src_ref[0]
    s_iota = jax.lax.broadcasted_iota(jnp.int32, (_TCHUNK, Ns), 1)
    oh_s = (sidx == s_iota).astype(jnp.bfloat16)
    gathered = jnp.dot(oh_s, src.astype(jnp.bfloat16),
                       preferred_element_type=jnp.float32)
    g_iota = jax.lax.broadcasted_iota(jnp.int32, (_TCHUNK, GW), 1)
    oh_g = (ihw == g_iota).astype(jnp.bfloat16)
    acc = jax.lax.dot_general(oh_g, gathered.astype(jnp.bfloat16),
                              (((0,), (0,)), ((), ())),
                              preferred_element_type=jnp.float32)
    out_ref[0] += acc

    @pl.when(c == _NCHUNK - 1)
    def _norm():
        g = out_ref[0]
        cnt = g[:, C + 1:C + 2] + 1e-6
        out_ref[0] = g / cnt


def _token2map(sidx, ihw, src_ext):
    grid = (B, _NCHUNK)
    return pl.pallas_call(
        _t2m_body,
        grid=grid,
        in_specs=[
            pl.BlockSpec((1, _TCHUNK, 1), lambda b, c: (b, c, 0)),
            pl.BlockSpec((1, _TCHUNK, 1), lambda b, c: (b, c, 0)),
            pl.BlockSpec((1, Ns, CE), lambda b, c: (b, 0, 0)),
        ],
        out_specs=pl.BlockSpec((1, GW, CE), lambda b, c: (b, 0, 0)),
        out_shape=jax.ShapeDtypeStruct((B, GW, CE), jnp.float32),
    )(sidx, ihw, src_ext)


def _mm_body(scale, x_ref, w_ref, b_ref, out_ref):
    x = x_ref[0]
    y = jnp.dot(x, w_ref[...], preferred_element_type=jnp.float32)
    y = y + b_ref[...]
    if scale != 1.0:
        y = y * scale
    out_ref[0] = y


def _matmul(x, w, b, mblk, scale=1.0):
    Bx, M, K = x.shape
    Nc = w.shape[1]
    grid = (Bx, M // mblk)
    return pl.pallas_call(
        functools.partial(_mm_body, scale),
        grid=grid,
        in_specs=[
            pl.BlockSpec((1, mblk, K), lambda b_, m: (b_, m, 0)),
            pl.BlockSpec((K, Nc), lambda b_, m: (0, 0)),
            pl.BlockSpec((1, Nc), lambda b_, m: (0, 0)),
        ],
        out_specs=pl.BlockSpec((1, mblk, Nc), lambda b_, m: (b_, m, 0)),
        out_shape=jax.ShapeDtypeStruct((Bx, M, Nc), jnp.float32),
    )(x, w, b)


def _sort_body(widx_ref, lt_ref, sut_ref, pos_ref):
    w1 = widx_ref[0]
    w_iota = jax.lax.broadcasted_iota(jnp.int32, (N, WPAD), 1)
    oh = (w1 == w_iota)
    ohb = oh.astype(jnp.bfloat16)
    pref = jnp.dot(lt_ref[...], ohb, preferred_element_type=jnp.float32)
    counts = pref[N - 1:N, :]
    offs = jnp.dot(counts, sut_ref[...],
                   precision=jax.lax.Precision.HIGHEST,
                   preferred_element_type=jnp.float32)
    pick = jnp.where(oh, pref + offs - 1.0, 0.0)
    pos_ref[0] = jnp.sum(pick, axis=1, keepdims=True).astype(jnp.int32)


def _sort_perm(widx):
    lt = jnp.asarray(np.tril(np.ones((N, N), np.float32)).astype(np.float32)
                     ).astype(jnp.bfloat16)
    sut = jnp.asarray(np.triu(np.ones((WPAD, WPAD), np.float32), 1))
    return pl.pallas_call(
        _sort_body,
        grid=(B,),
        in_specs=[
            pl.BlockSpec((1, N, 1), lambda b: (b, 0, 0)),
            pl.BlockSpec((N, N), lambda b: (0, 0)),
            pl.BlockSpec((WPAD, WPAD), lambda b: (0, 0)),
        ],
        out_specs=pl.BlockSpec((1, N, 1), lambda b: (b, 0, 0)),
        out_shape=jax.ShapeDtypeStruct((B, N, 1), jnp.int32),
    )(widx, lt, sut)


_PBLK = 512


def _perm_body(pos_ref, qa_ref, out_ref):
    pb = pl.program_id(1)
    pos = pos_ref[0]
    p_iota = jax.lax.broadcasted_iota(jnp.int32, (_PBLK, N), 0) + pb * _PBLK
    psort = (pos == p_iota).astype(jnp.bfloat16)
    qa = qa_ref[0].astype(jnp.bfloat16)
    out_ref[0] = jnp.dot(psort, qa, preferred_element_type=jnp.float32)


def _perm_q(pos_row, q_aug):
    return pl.pallas_call(
        _perm_body,
        grid=(B, N // _PBLK),
        in_specs=[
            pl.BlockSpec((1, 1, N), lambda b, p: (b, 0, 0)),
            pl.BlockSpec((1, N, C + 1), lambda b, p: (b, 0, 0)),
        ],
        out_specs=pl.BlockSpec((1, _PBLK, C + 1), lambda b, p: (b, p, 0)),
        out_shape=jax.ShapeDtypeStruct((B, N, C + 1), jnp.float32),
    )(pos_row, q_aug)


_ABLK = 256
_KTILE = 1024
_WPT = _KTILE // SLOTS


def _attn_body(q_ref, k_ref, v_ref, conf_ref, ws_ref, wslot_ref, out_ref):
    q = q_ref[0, 0].astype(jnp.bfloat16)
    ws = ws_ref[0]
    tlo = jnp.min(ws) // _WPT
    thi = jnp.max(ws) // _WPT

    def body(t, carry):
        o, s = carry
        kt = k_ref[0, 0, pl.ds(t * _KTILE, _KTILE), :].astype(jnp.bfloat16)
        vt = v_ref[0, 0, pl.ds(t * _KTILE, _KTILE), :].astype(jnp.bfloat16)
        cf = conf_ref[0, :, pl.ds(t * _KTILE, _KTILE)]
        wsl = wslot_ref[0, :, pl.ds(t * _KTILE, _KTILE)]
        logits = jax.lax.dot_general(q, kt, (((1,), (1,)), ((), ())),
                                     preferred_element_type=jnp.float32)
        lg = jnp.where(wsl == ws, logits + cf, jnp.float32(-jnp.inf))
        p = jnp.exp(lg)
        s = s + jnp.sum(p, axis=1, keepdims=True)
        o = o + jnp.dot(p.astype(jnp.bfloat16), vt,
                        preferred_element_type=jnp.float32)
        return o, s

    o0 = jnp.zeros((_ABLK, HD), jnp.float32)
    s0 = jnp.zeros((_ABLK, 1), jnp.float32)
    o, s = jax.lax.fori_loop(tlo, thi + 1, body, (o0, s0))
    out_ref[0, 0] = o / s


def _attention(q4s, k4w, v4w, confw, ws):
    grid = (B, NUM_HEADS, N // _ABLK)
    return pl.pallas_call(
        _attn_body,
        grid=grid,
        in_specs=[
            pl.BlockSpec((1, 1, _ABLK, HD), lambda b, h, n: (b, h, n, 0)),
            pl.BlockSpec((1, 1, GW, HD), lambda b, h, n: (b, h, 0, 0)),
            pl.BlockSpec((1, 1, GW, HD), lambda b, h, n: (b, h, 0, 0)),
            pl.BlockSpec((1, 1, GW), lambda b, h, n: (b, 0, 0)),
            pl.BlockSpec((1, _ABLK, 1), lambda b, h, n: (b, n, 0)),
            pl.BlockSpec((1, 1, GW), lambda b, h, n: (0, 0, 0)),
        ],
        out_specs=pl.BlockSpec((1, 1, _ABLK, HD), lambda b, h, n: (b, h, n, 0)),
        out_shape=jax.ShapeDtypeStruct((B, NUM_HEADS, N, HD), jnp.float32),
    )(q4s, k4w, v4w, confw, ws, jnp.asarray(_win_of_slot()))


def _unperm_body(pos_ref, att_ref, w_ref, b_ref, out_ref):
    posc = pos_ref[0]
    p_iota = jax.lax.broadcasted_iota(jnp.int32, (_PBLK, N), 1)
    pinv = (posc == p_iota).astype(jnp.bfloat16)
    a = jnp.dot(pinv, att_ref[0].astype(jnp.bfloat16),
                preferred_element_type=jnp.float32)
    y = jnp.dot(a.astype(jnp.bfloat16), w_ref[...].astype(jnp.bfloat16),
                preferred_element_type=jnp.float32)
    out_ref[0] = y + b_ref[...]


def _unperm_proj(pos, att_s, w, b):
    return pl.pallas_call(
        _unperm_body,
        grid=(B, N // _PBLK),
        in_specs=[
            pl.BlockSpec((1, _PBLK, 1), lambda b_, p: (b_, p, 0)),
            pl.BlockSpec((1, N, C), lambda b_, p: (b_, 0, 0)),
            pl.BlockSpec((C, C), lambda b_, p: (0, 0)),
            pl.BlockSpec((1, C), lambda b_, p: (0, 0)),
        ],
        out_specs=pl.BlockSpec((1, _PBLK, C), lambda b_, p: (b_, p, 0)),
        out_shape=jax.ShapeDtypeStruct((B, N, C), jnp.float32),
    )(pos, att_s, w, b)


def kernel(tar_x, tar_loc_orig, tar_idx_agg, tar_agg_weight, src_x,
           src_idx_agg, src_conf, map_h, map_w, Wq, bq, Wkv, bkv, Wp, bp):
    whf = jnp.stack([map_w, map_h]).astype(jnp.float32)

    loc = tar_loc_orig
    xy = 0.5 * (loc + 1.0) * whf[None, None, :] - 0.5
    xg = jnp.clip(jnp.round(xy[..., 0]).astype(jnp.int32), 0, W - 1)
    yg = jnp.clip(jnp.round(xy[..., 1]).astype(jnp.int32), 0, H - 1)
    idx_tmp = ((yg + PAD_OFF) // HWW) * NH + (xg + PAD_OFF) // HWW

    locc = jnp.clip(loc, -1.0, 1.0)
    locc = 0.5 * (locc + 1.0) * whf[None, None, :] - 0.5
    lx = jnp.clip(jnp.round(locc[..., 0]).astype(jnp.int32), 0, W - 1)
    ly = jnp.clip(jnp.round(locc[..., 1]).astype(jnp.int32), 0, H - 1)
    idx_hw = lx + ly * W
    islot = jnp.take(jnp.asarray(_slot_of_cell()), idx_hw)

    widx = _route(idx_tmp.reshape(B, N0, 1),
                  tar_idx_agg.astype(jnp.int32).reshape(B, 1, N0),
                  tar_agg_weight)

    src_ext = jnp.concatenate(
        [src_x, src_conf, jnp.ones((B, Ns, 1), jnp.float32),
         jnp.zeros((B, Ns, CE - C - 2), jnp.float32)], axis=-1)
    gridm = _token2map(src_idx_agg.astype(jnp.int32).reshape(B, N0, 1),
                       islot.reshape(B, N0, 1), src_ext)
    gx = gridm[..., :C]
    confw = gridm[..., C].reshape(B, 1, GW)

    scale = HD ** (-0.5)
    q = _matmul(tar_x, Wq, bq.reshape(1, C), 512, scale=scale)
    kv = _matmul(gx, Wkv, bkv.reshape(1, 2 * C), 512)

    pos = _sort_perm(widx)
    q_aug = jnp.concatenate([q, widx.astype(jnp.float32)], axis=-1)
    qs_aug = _perm_q(pos.reshape(B, 1, N), q_aug)
    q_s = qs_aug[..., :C]
    ws = qs_aug[..., C].astype(jnp.int32).reshape(B, N, 1)
    q4s = q_s.reshape(B, N, NUM_HEADS, HD).transpose(0, 2, 1, 3)

    k4w = kv[..., :C].reshape(B, GW, NUM_HEADS, HD).transpose(0, 2, 1, 3)
    v4w = kv[..., C:].reshape(B, GW, NUM_HEADS, HD).transpose(0, 2, 1, 3)

    att4s = _attention(q4s, k4w, v4w, confw, ws)
    att_s = att4s.transpose(0, 2, 1, 3).reshape(B, N, C)

    return _unperm_proj(pos, att_s, Wp, bp.reshape(1, C))


# speedup vs baseline: 1.2459x; 1.2459x over previous
import functools

import jax
import jax.numpy as jnp
import numpy as np
from jax.experimental import pallas as pl

B, N, C = 4, 2048, 192
N0, Ns = 4096, 2048
H, W = 64, 64
NUM_HEADS = 8
HD = C // NUM_HEADS
HWW = 7
NH = 10
PAD_OFF = 3
G = H * W
NW = NH * NH
WPAD = 128
CE = 256


def _win_of_grid():
    t = np.arange(G)
    y, x = t // W, t % W
    w = ((y + PAD_OFF) // HWW) * NH + (x + PAD_OFF) // HWW
    return w.astype(np.int32).reshape(1, 1, G)


_RBLK = 256


def _route_body(idxw_ref, agg_ref, aw_ref, out_ref):
    nb = pl.program_id(1)
    agg = agg_ref[0]
    aw = aw_ref[0]
    n_iota = jax.lax.broadcasted_iota(jnp.int32, (_RBLK, N0), 0) + nb * _RBLK
    oh_n = (agg == n_iota).astype(jnp.float32)
    iw = idxw_ref[0]
    w_iota = jax.lax.broadcasted_iota(jnp.int32, (N0, WPAD), 1)
    wv = jnp.where(iw == w_iota, aw, 0.0)
    votes = jax.lax.dot_general(oh_n, wv, (((1,), (0,)), ((), ())),
                                precision=jax.lax.Precision.HIGHEST,
                                preferred_element_type=jnp.float32)
    m = jnp.max(votes, axis=1, keepdims=True)
    cand = jnp.where(votes == m,
                     jax.lax.broadcasted_iota(jnp.int32, (_RBLK, WPAD), 1),
                     jnp.int32(2 ** 30))
    out_ref[0] = jnp.min(cand, axis=1, keepdims=True)


def _route(idx_tmp, agg, aw):
    grid = (B, N // _RBLK)
    return pl.pallas_call(
        _route_body,
        grid=grid,
        in_specs=[
            pl.BlockSpec((1, N0, 1), lambda b, n: (b, 0, 0)),
            pl.BlockSpec((1, 1, N0), lambda b, n: (b, 0, 0)),
            pl.BlockSpec((1, N0, 1), lambda b, n: (b, 0, 0)),
        ],
        out_specs=pl.BlockSpec((1, _RBLK, 1), lambda b, n: (b, n, 0)),
        out_shape=jax.ShapeDtypeStruct((B, N, 1), jnp.int32),
    )(idx_tmp, agg, aw)


_TCHUNK = 512
_NCHUNK = N0 // _TCHUNK


def _t2m_body(sidx_ref, ihw_ref, src_ref, out_ref):
    c = pl.program_id(1)

    @pl.when(c == 0)
    def _init():
        out_ref[0] = jnp.zeros((G, CE), jnp.float32)

    sidx = sidx_ref[0]
    ihw = ihw_ref[0]
    src = src_ref[0]
    s_iota = jax.lax.broadcasted_iota(jnp.int32, (_TCHUNK, Ns), 1)
    oh_s = (sidx == s_iota).astype(jnp.float32)
    gathered = jnp.dot(oh_s, src, preferred_element_type=jnp.float32)
    g_iota = jax.lax.broadcasted_iota(jnp.int32, (_TCHUNK, G), 1)
    oh_g = (ihw == g_iota).astype(jnp.float32)
    acc = jax.lax.dot_general(oh_g, gathered, (((0,), (0,)), ((), ())),
                              preferred_element_type=jnp.float32)
    out_ref[0] += acc

    @pl.when(c == _NCHUNK - 1)
    def _norm():
        g = out_ref[0]
        cnt = g[:, C + 1:C + 2] + 1e-6
        out_ref[0] = g / cnt


def _token2map(sidx, ihw, src_ext):
    grid = (B, _NCHUNK)
    return pl.pallas_call(
        _t2m_body,
        grid=grid,
        in_specs=[
            pl.BlockSpec((1, _TCHUNK, 1), lambda b, c: (b, c, 0)),
            pl.BlockSpec((1, _TCHUNK, 1), lambda b, c: (b, c, 0)),
            pl.BlockSpec((1, Ns, CE), lambda b, c: (b, 0, 0)),
        ],
        out_specs=pl.BlockSpec((1, G, CE), lambda b, c: (b, 0, 0)),
        out_shape=jax.ShapeDtypeStruct((B, G, CE), jnp.float32),
    )(sidx, ihw, src_ext)


def _mm_body(scale, x_ref, w_ref, b_ref, out_ref):
    x = x_ref[0]
    y = jnp.dot(x, w_ref[...], preferred_element_type=jnp.float32)
    y = y + b_ref[...]
    if scale != 1.0:
        y = y * scale
    out_ref[0] = y


def _matmul(x, w, b, mblk, scale=1.0):
    Bx, M, K = x.shape
    Nc = w.shape[1]
    grid = (Bx, M // mblk)
    return pl.pallas_call(
        functools.partial(_mm_body, scale),
        grid=grid,
        in_specs=[
            pl.BlockSpec((1, mblk, K), lambda b_, m: (b_, m, 0)),
            pl.BlockSpec((K, Nc), lambda b_, m: (0, 0)),
            pl.BlockSpec((1, Nc), lambda b_, m: (0, 0)),
        ],
        out_specs=pl.BlockSpec((1, mblk, Nc), lambda b_, m: (b_, m, 0)),
        out_shape=jax.ShapeDtypeStruct((Bx, M, Nc), jnp.float32),
    )(x, w, b)


_ABLK = 256


def _attn_body(q_ref, k_ref, v_ref, conf_ref, widx_ref, wot_ref, out_ref):
    q = q_ref[0, 0]
    k = k_ref[0, 0]
    v = v_ref[0, 0]
    conf = conf_ref[0]
    widx = widx_ref[0]
    wot = wot_ref[0]
    logits = jax.lax.dot_general(q.astype(jnp.bfloat16),
                                 k.astype(jnp.bfloat16),
                                 (((1,), (1,)), ((), ())),
                                 preferred_element_type=jnp.float32)
    lg = jnp.where(wot == widx, (logits + conf).astype(jnp.bfloat16),
                   jnp.bfloat16(-jnp.inf))
    p = jnp.exp(lg)
    s = jnp.sum(p.astype(jnp.float32), axis=1, keepdims=True)
    o = jnp.dot(p, v.astype(jnp.bfloat16),
                preferred_element_type=jnp.float32)
    out_ref[0, 0] = o / s


def _attention(q4, k4, v4, conf, widx):
    grid = (B, NUM_HEADS, N // _ABLK)
    return pl.pallas_call(
        _attn_body,
        grid=grid,
        in_specs=[
            pl.BlockSpec((1, 1, _ABLK, HD), lambda b, h, n: (b, h, n, 0)),
            pl.BlockSpec((1, 1, G, HD), lambda b, h, n: (b, h, 0, 0)),
            pl.BlockSpec((1, 1, G, HD), lambda b, h, n: (b, h, 0, 0)),
            pl.BlockSpec((1, 1, G), lambda b, h, n: (b, 0, 0)),
            pl.BlockSpec((1, _ABLK, 1), lambda b, h, n: (b, n, 0)),
            pl.BlockSpec((1, 1, G), lambda b, h, n: (0, 0, 0)),
        ],
        out_specs=pl.BlockSpec((1, 1, _ABLK, HD), lambda b, h, n: (b, h, n, 0)),
        out_shape=jax.ShapeDtypeStruct((B, NUM_HEADS, N, HD), jnp.float32),
    )(q4, k4, v4, conf, widx, jnp.asarray(_win_of_grid()))


def kernel(tar_x, tar_loc_orig, tar_idx_agg, tar_agg_weight, src_x,
           src_idx_agg, src_conf, map_h, map_w, Wq, bq, Wkv, bkv, Wp, bp):
    whf = jnp.stack([map_w, map_h]).astype(jnp.float32)

    loc = tar_loc_orig
    xy = 0.5 * (loc + 1.0) * whf[None, None, :] - 0.5
    xg = jnp.clip(jnp.round(xy[..., 0]).astype(jnp.int32), 0, W - 1)
    yg = jnp.clip(jnp.round(xy[..., 1]).astype(jnp.int32), 0, H - 1)
    idx_tmp = ((yg + PAD_OFF) // HWW) * NH + (xg + PAD_OFF) // HWW

    locc = jnp.clip(loc, -1.0, 1.0)
    locc = 0.5 * (locc + 1.0) * whf[None, None, :] - 0.5
    lx = jnp.clip(jnp.round(locc[..., 0]).astype(jnp.int32), 0, W - 1)
    ly = jnp.clip(jnp.round(locc[..., 1]).astype(jnp.int32), 0, H - 1)
    idx_hw = lx + ly * W

    widx = _route(idx_tmp.reshape(B, N0, 1),
                  tar_idx_agg.astype(jnp.int32).reshape(B, 1, N0),
                  tar_agg_weight)

    src_ext = jnp.concatenate(
        [src_x, src_conf, jnp.ones((B, Ns, 1), jnp.float32),
         jnp.zeros((B, Ns, CE - C - 2), jnp.float32)], axis=-1)
    gridm = _token2map(src_idx_agg.astype(jnp.int32).reshape(B, N0, 1),
                       idx_hw.reshape(B, N0, 1), src_ext)
    gx = gridm[..., :C]
    conf = gridm[..., C].reshape(B, 1, G)

    scale = HD ** (-0.5)
    q = _matmul(tar_x, Wq, bq.reshape(1, C), 512, scale=scale)
    kv = _matmul(gx, Wkv, bkv.reshape(1, 2 * C), 512)
    q4 = q.reshape(B, N, NUM_HEADS, HD).transpose(0, 2, 1, 3)
    k4 = kv[..., :C].reshape(B, G, NUM_HEADS, HD).transpose(0, 2, 1, 3)
    v4 = kv[..., C:].reshape(B, G, NUM_HEADS, HD).transpose(0, 2, 1, 3)

    att4 = _attention(q4, k4, v4, conf, widx)
    att = att4.transpose(0, 2, 1, 3).reshape(B, N, C)

    return _matmul(att, Wp, bp.reshape(1, C), 512)


# speedup vs baseline: 1.4548x; 1.1677x over previous
import functools

import jax
import jax.numpy as jnp
import numpy as np
from jax.experimental import pallas as pl

B, N, C = 4, 2048, 192
N0, Ns = 4096, 2048
H, W = 64, 64
NUM_HEADS = 8
HD = C // NUM_HEADS
HWW = 7
NH = 10
PAD_OFF = 3
G = H * W
NW = NH * NH
WPAD = 128
CE = 256


def _win_of_grid():
    t = np.arange(G)
    y, x = t // W, t % W
    w = ((y + PAD_OFF) // HWW) * NH + (x + PAD_OFF) // HWW
    return w.astype(np.int32).reshape(1, 1, G)


_RBLK = 256


def _route_body(idxw_ref, agg_ref, aw_ref, out_ref):
    nb = pl.program_id(1)
    agg = agg_ref[0]
    aw = aw_ref[0]
    n_iota = jax.lax.broadcasted_iota(jnp.int32, (_RBLK, N0), 0) + nb * _RBLK
    oh_n = (agg == n_iota).astype(jnp.float32)
    iw = idxw_ref[0]
    w_iota = jax.lax.broadcasted_iota(jnp.int32, (N0, WPAD), 1)
    wv = jnp.where(iw == w_iota, aw, 0.0)
    votes = jax.lax.dot_general(oh_n, wv, (((1,), (0,)), ((), ())),
                                precision=jax.lax.Precision.HIGHEST,
                                preferred_element_type=jnp.float32)
    m = jnp.max(votes, axis=1, keepdims=True)
    cand = jnp.where(votes == m,
                     jax.lax.broadcasted_iota(jnp.int32, (_RBLK, WPAD), 1),
                     jnp.int32(2 ** 30))
    out_ref[0] = jnp.min(cand, axis=1, keepdims=True)


def _route(idx_tmp, agg, aw):
    grid = (B, N // _RBLK)
    return pl.pallas_call(
        _route_body,
        grid=grid,
        in_specs=[
            pl.BlockSpec((1, N0, 1), lambda b, n: (b, 0, 0)),
            pl.BlockSpec((1, 1, N0), lambda b, n: (b, 0, 0)),
            pl.BlockSpec((1, N0, 1), lambda b, n: (b, 0, 0)),
        ],
        out_specs=pl.BlockSpec((1, _RBLK, 1), lambda b, n: (b, n, 0)),
        out_shape=jax.ShapeDtypeStruct((B, N, 1), jnp.int32),
    )(idx_tmp, agg, aw)


_TCHUNK = 512
_NCHUNK = N0 // _TCHUNK


def _t2m_body(sidx_ref, ihw_ref, src_ref, out_ref):
    c = pl.program_id(1)

    @pl.when(c == 0)
    def _init():
        out_ref[0] = jnp.zeros((G, CE), jnp.float32)

    sidx = sidx_ref[0]
    ihw = ihw_ref[0]
    src = src_ref[0]
    s_iota = jax.lax.broadcasted_iota(jnp.int32, (_TCHUNK, Ns), 1)
    oh_s = (sidx == s_iota).astype(jnp.float32)
    gathered = jnp.dot(oh_s, src, preferred_element_type=jnp.float32)
    g_iota = jax.lax.broadcasted_iota(jnp.int32, (_TCHUNK, G), 1)
    oh_g = (ihw == g_iota).astype(jnp.float32)
    acc = jax.lax.dot_general(oh_g, gathered, (((0,), (0,)), ((), ())),
                              preferred_element_type=jnp.float32)
    out_ref[0] += acc

    @pl.when(c == _NCHUNK - 1)
    def _norm():
        g = out_ref[0]
        cnt = g[:, C + 1:C + 2] + 1e-6
        out_ref[0] = g / cnt


def _token2map(sidx, ihw, src_ext):
    grid = (B, _NCHUNK)
    return pl.pallas_call(
        _t2m_body,
        grid=grid,
        in_specs=[
            pl.BlockSpec((1, _TCHUNK, 1), lambda b, c: (b, c, 0)),
            pl.BlockSpec((1, _TCHUNK, 1), lambda b, c: (b, c, 0)),
            pl.BlockSpec((1, Ns, CE), lambda b, c: (b, 0, 0)),
        ],
        out_specs=pl.BlockSpec((1, G, CE), lambda b, c: (b, 0, 0)),
        out_shape=jax.ShapeDtypeStruct((B, G, CE), jnp.float32),
    )(sidx, ihw, src_ext)


def _mm_body(scale, x_ref, w_ref, b_ref, out_ref):
    x = x_ref[0]
    y = jnp.dot(x, w_ref[...], preferred_element_type=jnp.float32)
    y = y + b_ref[...]
    if scale != 1.0:
        y = y * scale
    out_ref[0] = y


def _matmul(x, w, b, mblk, scale=1.0):
    Bx, M, K = x.shape
    Nc = w.shape[1]
    grid = (Bx, M // mblk)
    return pl.pallas_call(
        functools.partial(_mm_body, scale),
        grid=grid,
        in_specs=[
            pl.BlockSpec((1, mblk, K), lambda b_, m: (b_, m, 0)),
            pl.BlockSpec((K, Nc), lambda b_, m: (0, 0)),
            pl.BlockSpec((1, Nc), lambda b_, m: (0, 0)),
        ],
        out_specs=pl.BlockSpec((1, mblk, Nc), lambda b_, m: (b_, m, 0)),
        out_shape=jax.ShapeDtypeStruct((Bx, M, Nc), jnp.float32),
    )(x, w, b)


_ABLK = 512


def _attn_body(q_ref, k_ref, v_ref, conf_ref, widx_ref, wot_ref, out_ref):
    q = q_ref[0, 0]
    k = k_ref[0, 0]
    v = v_ref[0, 0]
    conf = conf_ref[0]
    widx = widx_ref[0]
    wot = wot_ref[0]
    logits = jax.lax.dot_general(q.astype(jnp.bfloat16),
                                 k.astype(jnp.bfloat16),
                                 (((1,), (1,)), ((), ())),
                                 preferred_element_type=jnp.float32)
    lg = jnp.where(wot == widx, (logits + conf).astype(jnp.bfloat16),
                   jnp.bfloat16(-jnp.inf))
    p = jnp.exp(lg)
    s = jnp.sum(p.astype(jnp.float32), axis=1, keepdims=True)
    o = jnp.dot(p, v.astype(jnp.bfloat16),
                preferred_element_type=jnp.float32)
    out_ref[0, 0] = o / s


def _attention(q4, k4, v4, conf, widx):
    grid = (B, NUM_HEADS, N // _ABLK)
    return pl.pallas_call(
        _attn_body,
        grid=grid,
        in_specs=[
            pl.BlockSpec((1, 1, _ABLK, HD), lambda b, h, n: (b, h, n, 0)),
            pl.BlockSpec((1, 1, G, HD), lambda b, h, n: (b, h, 0, 0)),
            pl.BlockSpec((1, 1, G, HD), lambda b, h, n: (b, h, 0, 0)),
            pl.BlockSpec((1, 1, G), lambda b, h, n: (b, 0, 0)),
            pl.BlockSpec((1, _ABLK, 1), lambda b, h, n: (b, n, 0)),
            pl.BlockSpec((1, 1, G), lambda b, h, n: (0, 0, 0)),
        ],
        out_specs=pl.BlockSpec((1, 1, _ABLK, HD), lambda b, h, n: (b, h, n, 0)),
        out_shape=jax.ShapeDtypeStruct((B, NUM_HEADS, N, HD), jnp.float32),
    )(q4, k4, v4, conf, widx, jnp.asarray(_win_of_grid()))


def kernel(tar_x, tar_loc_orig, tar_idx_agg, tar_agg_weight, src_x,
           src_idx_agg, src_conf, map_h, map_w, Wq, bq, Wkv, bkv, Wp, bp):
    whf = jnp.stack([map_w, map_h]).astype(jnp.float32)

    loc = tar_loc_orig
    xy = 0.5 * (loc + 1.0) * whf[None, None, :] - 0.5
    xg = jnp.clip(jnp.round(xy[..., 0]).astype(jnp.int32), 0, W - 1)
    yg = jnp.clip(jnp.round(xy[..., 1]).astype(jnp.int32), 0, H - 1)
    idx_tmp = ((yg + PAD_OFF) // HWW) * NH + (xg + PAD_OFF) // HWW

    locc = jnp.clip(loc, -1.0, 1.0)
    locc = 0.5 * (locc + 1.0) * whf[None, None, :] - 0.5
    lx = jnp.clip(jnp.round(locc[..., 0]).astype(jnp.int32), 0, W - 1)
    ly = jnp.clip(jnp.round(locc[..., 1]).astype(jnp.int32), 0, H - 1)
    idx_hw = lx + ly * W

    widx = _route(idx_tmp.reshape(B, N0, 1),
                  tar_idx_agg.astype(jnp.int32).reshape(B, 1, N0),
                  tar_agg_weight)

    src_ext = jnp.concatenate(
        [src_x, src_conf, jnp.ones((B, Ns, 1), jnp.float32),
         jnp.zeros((B, Ns, CE - C - 2), jnp.float32)], axis=-1)
    gridm = _token2map(src_idx_agg.astype(jnp.int32).reshape(B, N0, 1),
                       idx_hw.reshape(B, N0, 1), src_ext)
    gx = gridm[..., :C]
    conf = gridm[..., C].reshape(B, 1, G)

    scale = HD ** (-0.5)
    q = _matmul(tar_x, Wq, bq.reshape(1, C), 512, scale=scale)
    kv = _matmul(gx, Wkv, bkv.reshape(1, 2 * C), 512)
    q4 = q.reshape(B, N, NUM_HEADS, HD).transpose(0, 2, 1, 3)
    k4 = kv[..., :C].reshape(B, G, NUM_HEADS, HD).transpose(0, 2, 1, 3)
    v4 = kv[..., C:].reshape(B, G, NUM_HEADS, HD).transpose(0, 2, 1, 3)

    att4 = _attention(q4, k4, v4, conf, widx)
    att = att4.transpose(0, 2, 1, 3).reshape(B, N, C)

    return _matmul(att, Wp, bp.reshape(1, C), 512)


# speedup vs baseline: 1.4821x; 1.0188x over previous
import functools

import jax
import jax.numpy as jnp
import numpy as np
from jax.experimental import pallas as pl

B, N, C = 4, 2048, 192
N0, Ns = 4096, 2048
H, W = 64, 64
NUM_HEADS = 8
HD = C // NUM_HEADS
HWW = 7
NH = 10
PAD_OFF = 3
G = H * W
NW = NH * NH
WPAD = 128
CE = 256


def _win_of_grid():
    t = np.arange(G)
    y, x = t // W, t % W
    w = ((y + PAD_OFF) // HWW) * NH + (x + PAD_OFF) // HWW
    return w.astype(np.int32).reshape(1, 1, G)


_RBLK = 256


def _route_body(idxw_ref, agg_ref, aw_ref, out_ref):
    nb = pl.program_id(1)
    agg = agg_ref[0]
    aw = aw_ref[0]
    n_iota = jax.lax.broadcasted_iota(jnp.int32, (_RBLK, N0), 0) + nb * _RBLK
    oh_n = (agg == n_iota).astype(jnp.float32)
    iw = idxw_ref[0]
    w_iota = jax.lax.broadcasted_iota(jnp.int32, (N0, WPAD), 1)
    wv = jnp.where(iw == w_iota, aw, 0.0)
    votes = jax.lax.dot_general(oh_n, wv, (((1,), (0,)), ((), ())),
                                precision=jax.lax.Precision.HIGHEST,
                                preferred_element_type=jnp.float32)
    m = jnp.max(votes, axis=1, keepdims=True)
    cand = jnp.where(votes == m,
                     jax.lax.broadcasted_iota(jnp.int32, (_RBLK, WPAD), 1),
                     jnp.int32(2 ** 30))
    out_ref[0] = jnp.min(cand, axis=1, keepdims=True)


def _route(idx_tmp, agg, aw):
    grid = (B, N // _RBLK)
    return pl.pallas_call(
        _route_body,
        grid=grid,
        in_specs=[
            pl.BlockSpec((1, N0, 1), lambda b, n: (b, 0, 0)),
            pl.BlockSpec((1, 1, N0), lambda b, n: (b, 0, 0)),
            pl.BlockSpec((1, N0, 1), lambda b, n: (b, 0, 0)),
        ],
        out_specs=pl.BlockSpec((1, _RBLK, 1), lambda b, n: (b, n, 0)),
        out_shape=jax.ShapeDtypeStruct((B, N, 1), jnp.int32),
    )(idx_tmp, agg, aw)


_TCHUNK = 512
_NCHUNK = N0 // _TCHUNK


def _t2m_body(sidx_ref, ihw_ref, src_ref, out_ref):
    c = pl.program_id(1)

    @pl.when(c == 0)
    def _init():
        out_ref[0] = jnp.zeros((G, CE), jnp.float32)

    sidx = sidx_ref[0]
    ihw = ihw_ref[0]
    src = src_ref[0]
    s_iota = jax.lax.broadcasted_iota(jnp.int32, (_TCHUNK, Ns), 1)
    oh_s = (sidx == s_iota).astype(jnp.float32)
    gathered = jnp.dot(oh_s, src, preferred_element_type=jnp.float32)
    g_iota = jax.lax.broadcasted_iota(jnp.int32, (_TCHUNK, G), 1)
    oh_g = (ihw == g_iota).astype(jnp.float32)
    acc = jax.lax.dot_general(oh_g, gathered, (((0,), (0,)), ((), ())),
                              preferred_element_type=jnp.float32)
    out_ref[0] += acc

    @pl.when(c == _NCHUNK - 1)
    def _norm():
        g = out_ref[0]
        cnt = g[:, C + 1:C + 2] + 1e-6
        out_ref[0] = g / cnt


def _token2map(sidx, ihw, src_ext):
    grid = (B, _NCHUNK)
    return pl.pallas_call(
        _t2m_body,
        grid=grid,
        in_specs=[
            pl.BlockSpec((1, _TCHUNK, 1), lambda b, c: (b, c, 0)),
            pl.BlockSpec((1, _TCHUNK, 1), lambda b, c: (b, c, 0)),
            pl.BlockSpec((1, Ns, CE), lambda b, c: (b, 0, 0)),
        ],
        out_specs=pl.BlockSpec((1, G, CE), lambda b, c: (b, 0, 0)),
        out_shape=jax.ShapeDtypeStruct((B, G, CE), jnp.float32),
    )(sidx, ihw, src_ext)


def _mm_body(scale, x_ref, w_ref, b_ref, out_ref):
    x = x_ref[0]
    y = jnp.dot(x, w_ref[...], preferred_element_type=jnp.float32)
    y = y + b_ref[...]
    if scale != 1.0:
        y = y * scale
    out_ref[0] = y


def _matmul(x, w, b, mblk, scale=1.0):
    Bx, M, K = x.shape
    Nc = w.shape[1]
    grid = (Bx, M // mblk)
    return pl.pallas_call(
        functools.partial(_mm_body, scale),
        grid=grid,
        in_specs=[
            pl.BlockSpec((1, mblk, K), lambda b_, m: (b_, m, 0)),
            pl.BlockSpec((K, Nc), lambda b_, m: (0, 0)),
            pl.BlockSpec((1, Nc), lambda b_, m: (0, 0)),
        ],
        out_specs=pl.BlockSpec((1, mblk, Nc), lambda b_, m: (b_, m, 0)),
        out_shape=jax.ShapeDtypeStruct((Bx, M, Nc), jnp.float32),
    )(x, w, b)


_ABLK = 1024


def _attn_body(q_ref, k_ref, v_ref, conf_ref, widx_ref, wot_ref, out_ref):
    q = q_ref[0, 0]
    k = k_ref[0, 0]
    v = v_ref[0, 0]
    conf = conf_ref[0]
    widx = widx_ref[0]
    wot = wot_ref[0]
    logits = jax.lax.dot_general(q.astype(jnp.bfloat16),
                                 k.astype(jnp.bfloat16),
                                 (((1,), (1,)), ((), ())),
                                 preferred_element_type=jnp.float32)
    lg = jnp.where(wot == widx, (logits + conf).astype(jnp.bfloat16),
                   jnp.bfloat16(-jnp.inf))
    p = jnp.exp(lg)
    s = jnp.sum(p.astype(jnp.float32), axis=1, keepdims=True)
    o = jnp.dot(p, v.astype(jnp.bfloat16),
                preferred_element_type=jnp.float32)
    out_ref[0, 0] = o / s


def _attention(q4, k4, v4, conf, widx):
    grid = (B, NUM_HEADS, N // _ABLK)
    return pl.pallas_call(
        _attn_body,
        grid=grid,
        in_specs=[
            pl.BlockSpec((1, 1, _ABLK, HD), lambda b, h, n: (b, h, n, 0)),
            pl.BlockSpec((1, 1, G, HD), lambda b, h, n: (b, h, 0, 0)),
            pl.BlockSpec((1, 1, G, HD), lambda b, h, n: (b, h, 0, 0)),
            pl.BlockSpec((1, 1, G), lambda b, h, n: (b, 0, 0)),
            pl.BlockSpec((1, _ABLK, 1), lambda b, h, n: (b, n, 0)),
            pl.BlockSpec((1, 1, G), lambda b, h, n: (0, 0, 0)),
        ],
        out_specs=pl.BlockSpec((1, 1, _ABLK, HD), lambda b, h, n: (b, h, n, 0)),
        out_shape=jax.ShapeDtypeStruct((B, NUM_HEADS, N, HD), jnp.float32),
    )(q4, k4, v4, conf, widx, jnp.asarray(_win_of_grid()))


def kernel(tar_x, tar_loc_orig, tar_idx_agg, tar_agg_weight, src_x,
           src_idx_agg, src_conf, map_h, map_w, Wq, bq, Wkv, bkv, Wp, bp):
    whf = jnp.stack([map_w, map_h]).astype(jnp.float32)

    loc = tar_loc_orig
    xy = 0.5 * (loc + 1.0) * whf[None, None, :] - 0.5
    xg = jnp.clip(jnp.round(xy[..., 0]).astype(jnp.int32), 0, W - 1)
    yg = jnp.clip(jnp.round(xy[..., 1]).astype(jnp.int32), 0, H - 1)
    idx_tmp = ((yg + PAD_OFF) // HWW) * NH + (xg + PAD_OFF) // HWW

    locc = jnp.clip(loc, -1.0, 1.0)
    locc = 0.5 * (locc + 1.0) * whf[None, None, :] - 0.5
    lx = jnp.clip(jnp.round(locc[..., 0]).astype(jnp.int32), 0, W - 1)
    ly = jnp.clip(jnp.round(locc[..., 1]).astype(jnp.int32), 0, H - 1)
    idx_hw = lx + ly * W

    widx = _route(idx_tmp.reshape(B, N0, 1),
                  tar_idx_agg.astype(jnp.int32).reshape(B, 1, N0),
                  tar_agg_weight)

    src_ext = jnp.concatenate(
        [src_x, src_conf, jnp.ones((B, Ns, 1), jnp.float32),
         jnp.zeros((B, Ns, CE - C - 2), jnp.float32)], axis=-1)
    gridm = _token2map(src_idx_agg.astype(jnp.int32).reshape(B, N0, 1),
                       idx_hw.reshape(B, N0, 1), src_ext)
    gx = gridm[..., :C]
    conf = gridm[..., C].reshape(B, 1, G)

    scale = HD ** (-0.5)
    q = _matmul(tar_x, Wq, bq.reshape(1, C), 512, scale=scale)
    kv = _matmul(gx, Wkv, bkv.reshape(1, 2 * C), 512)
    q4 = q.reshape(B, N, NUM_HEADS, HD).transpose(0, 2, 1, 3)
    k4 = kv[..., :C].reshape(B, G, NUM_HEADS, HD).transpose(0, 2, 1, 3)
    v4 = kv[..., C:].reshape(B, G, NUM_HEADS, HD).transpose(0, 2, 1, 3)

    att4 = _attention(q4, k4, v4, conf, widx)
    att = att4.transpose(0, 2, 1, 3).reshape(B, N, C)

    return _matmul(att, Wp, bp.reshape(1, C), 512)
